# Initial kernel scaffold; baseline (speedup 1.0000x reference)
import jax
import jax.numpy as jnp
from jax.experimental import pallas as pl


def kernel(user_fids, user_weighted_fids, user_weighted_fid_weights, fids, weighted_fids, weighted_fid_weights, day, table, W1, b1, W2, b2, W3, b3):
    raise NotImplementedError("write your pallas kernel here")


# speedup vs baseline: 49.3671x; 49.3671x over previous
import functools

import jax
import jax.numpy as jnp
from jax import lax
from jax.experimental import pallas as pl
from jax.experimental.pallas import tpu as pltpu
from jax.experimental.pallas import tpu_sc as plsc

FEATURE_BITS = 48
B = 4096
DIM = 64
VOCAB = 1000000
POW48_MOD = (1 << 48) % VOCAB

N_USER_SLOTS = 17
N_ITEM_SLOTS = 8
NSLOT = N_USER_SLOTS + N_ITEM_SLOTS

NF_UNW = 56
NF_W = 18
NF = 80
NW_PAD = 24

NC = 2
NS = 16
NWORKERS = NC * NS
LANES = 16

ROWS_PER_W = B // NWORKERS
G = 8
NCHUNK = ROWS_PER_W // G
FIDS_PER_CHUNK = G * NF
NIDX = FIDS_PER_CHUNK // 128
TRASH = G * NSLOT
PROWS = G * NSLOT + 8


def _sc_pool(hi, lo, w, table):
    mesh = plsc.VectorSubcoreMesh(core_axis_name="c", subcore_axis_name="s")

    @functools.partial(
        pl.kernel,
        out_type=jax.ShapeDtypeStruct((B * NSLOT, DIM), jnp.float32),
        mesh=mesh,
        scratch_types=[
            pltpu.VMEM((G, NF), jnp.int32),
            pltpu.VMEM((G, NF), jnp.int32),
            pltpu.VMEM((G, NW_PAD), jnp.float32),
            pltpu.VMEM((NIDX, 128), jnp.int32),
            pltpu.VMEM((NIDX, 128), jnp.int32),
            pltpu.VMEM((FIDS_PER_CHUNK, DIM), jnp.float32),
            pltpu.VMEM((PROWS, DIM), jnp.float32),
            pltpu.VMEM_SHARED((NS * PROWS, DIM), jnp.float32),
            pltpu.SemaphoreType.DMA,
        ],
        compiler_params=pltpu.CompilerParams(use_tc_tiling_on_sc=False),
    )
    def k(hi_hbm, lo_hbm, w_hbm, table_hbm, out_hbm,
          hi_v, lo_v, w_v, rows_v, dest_v, g_v, zeros_v, pool_sh, sem):
        sid = lax.axis_index("s")
        wid = sid * NC + lax.axis_index("c")
        row0 = wid * ROWS_PER_W
        pbase = sid * PROWS

        def zero(i, _):
            for t in range(DIM // LANES):
                zeros_v[i, pl.ds(t * LANES, LANES)] = jnp.zeros(
                    (LANES,), jnp.float32)
            return None

        lax.fori_loop(jnp.int32(0), jnp.int32(PROWS), zero, None)

        def chunk_body(c, _):
            base = row0 + c * G
            pltpu.sync_copy(hi_hbm.at[pl.ds(base, G)], hi_v)
            pltpu.sync_copy(lo_hbm.at[pl.ds(base, G)], lo_v)
            pltpu.sync_copy(w_hbm.at[pl.ds(base, G)], w_v)

            for g in range(G):
                for t in range(NF // LANES):
                    hi16 = hi_v[g, pl.ds(t * LANES, LANES)]
                    lo16 = lo_v[g, pl.ds(t * LANES, LANES)]
                    i32 = lambda v: jnp.int32(v)
                    slot = lax.shift_right_logical(hi16, i32(16))
                    row = (slot * i32(POW48_MOD) + lo16) % i32(VOCAB)
                    sidx = jnp.where(slot < i32(100), slot - i32(1),
                                     slot - i32(101 - N_USER_SLOTS))
                    dest = pbase + jnp.where(slot > i32(0),
                                             i32(g * NSLOT) + sidx, i32(TRASH))
                    q = g * NF + t * LANES
                    rows_v[q // 128, pl.ds(q % 128, LANES)] = row
                    dest_v[q // 128, pl.ds(q % 128, LANES)] = dest

            for j in range(NIDX):
                pltpu.async_copy(
                    table_hbm.at[rows_v.at[jnp.int32(j)]],
                    g_v.at[pl.ds(j * 128, 128)], sem).wait()

            for g in range(G):
                wa = w_v[g, pl.ds(0, LANES)]
                wb = w_v[g, pl.ds(NW_PAD - LANES, LANES)]
                for p in range(NW_PAD):
                    wgt = wa[p] if p < LANES else wb[p - (NW_PAD - LANES)]
                    q = g * NF + NF_UNW + p
                    for t in range(DIM // LANES):
                        g_v[q, pl.ds(t * LANES, LANES)] = (
                            g_v[q, pl.ds(t * LANES, LANES)] * wgt)

            pltpu.sync_copy(zeros_v, pool_sh.at[pl.ds(pbase, PROWS)])

            for j in range(NIDX):
                pltpu.async_copy(
                    g_v.at[pl.ds(j * 128, 128)],
                    pool_sh.at[dest_v.at[jnp.int32(j)]], sem, add=True).wait()

            pltpu.sync_copy(pool_sh.at[pl.ds(pbase, G * NSLOT)],
                            out_hbm.at[pl.ds(base * NSLOT, G * NSLOT)])
            return None

        lax.fori_loop(jnp.int32(0), jnp.int32(NCHUNK), chunk_body, None)

    return k(hi, lo, w, table)


def _mlp_kernel(x_ref, day_ref, w1_ref, b1_ref, w2_ref, b2_ref, w3_ref,
                b3_ref, out_ref):
    h = jnp.dot(x_ref[...], w1_ref[...], preferred_element_type=jnp.float32)
    h = jnp.maximum(h + b1_ref[...], 0.0)
    h = jnp.dot(h, w2_ref[...], preferred_element_type=jnp.float32)
    h = jnp.maximum(h + b2_ref[...], 0.0)
    o = jnp.dot(h, w3_ref[...], preferred_element_type=jnp.float32)
    o = o + b3_ref[...]
    cols = lax.broadcasted_iota(jnp.int32, o.shape, 1)
    sel = jnp.where(cols == day_ref[...], o, 0.0)
    out_ref[...] = jnp.sum(sel, axis=1, keepdims=True)


def _mlp(x, day, W1, b1, W2, b2, W3p, b3p):
    bm = 512
    grid = (B // bm,)
    return pl.pallas_call(
        _mlp_kernel,
        grid=grid,
        in_specs=[
            pl.BlockSpec((bm, W1.shape[0]), lambda i: (i, jnp.int32(0))),
            pl.BlockSpec((bm, 1), lambda i: (i, jnp.int32(0))),
            pl.BlockSpec(W1.shape, lambda i: (jnp.int32(0), jnp.int32(0))),
            pl.BlockSpec(b1.shape, lambda i: (jnp.int32(0), jnp.int32(0))),
            pl.BlockSpec(W2.shape, lambda i: (jnp.int32(0), jnp.int32(0))),
            pl.BlockSpec(b2.shape, lambda i: (jnp.int32(0), jnp.int32(0))),
            pl.BlockSpec(W3p.shape, lambda i: (jnp.int32(0), jnp.int32(0))),
            pl.BlockSpec(b3p.shape, lambda i: (jnp.int32(0), jnp.int32(0))),
        ],
        out_specs=pl.BlockSpec((bm, 1), lambda i: (i, jnp.int32(0))),
        out_shape=jax.ShapeDtypeStruct((B, 1), jnp.float32),
    )(x, day, W1, b1, W2, b2, W3p, b3p)


def kernel(user_fids, user_weighted_fids, user_weighted_fid_weights, fids,
           weighted_fids, weighted_fid_weights, day, table, W1, b1, W2, b2,
           W3, b3):
    fid_all = jnp.concatenate(
        [user_fids, fids, user_weighted_fids, weighted_fids,
         jnp.zeros((B, NF - 74), jnp.int64)], axis=1)
    pair = lax.bitcast_convert_type(fid_all, jnp.int32)
    lo = pair[..., 0]
    hi = pair[..., 1]
    w = jnp.concatenate(
        [user_weighted_fid_weights.astype(jnp.float32),
         weighted_fid_weights.astype(jnp.float32),
         jnp.zeros((B, NF - 74), jnp.float32)], axis=1)

    pooled = _sc_pool(hi, lo, w, table)
    x = pooled.reshape(B, NSLOT * DIM)

    W3p = jnp.pad(W3.astype(jnp.float32), ((0, 0), (0, 128 - W3.shape[1])))
    b3p = jnp.pad(b3.astype(jnp.float32), (0, 128 - b3.shape[0]))
    out = _mlp(x, day.astype(jnp.int32).reshape(B, 1),
               W1.astype(jnp.float32), b1.astype(jnp.float32).reshape(1, -1),
               W2.astype(jnp.float32), b2.astype(jnp.float32).reshape(1, -1),
               W3p, b3p.reshape(1, -1))
    return out
